# baseline (speedup 1.0000x reference)
import functools

import jax
import jax.numpy as jnp
from jax import lax
from jax.experimental import pallas as pl
from jax.experimental.pallas import tpu as pltpu
from jax.experimental.pallas import tpu_sc as plsc

_B, _S, _D = 4, 8192, 1024


_BLK_S = 512


def _add_block(x_ref, pos_ref, o_ref):
    o_ref[...] = x_ref[...] + pos_ref[...][None, :, :]


def _tc_kernel(x, pos):
    B, S, D = x.shape
    grid = (S // _BLK_S,)
    return pl.pallas_call(
        _add_block,
        grid=grid,
        in_specs=[
            pl.BlockSpec((B, _BLK_S, D), lambda s: (0, s, 0)),
            pl.BlockSpec((_BLK_S, D), lambda s: (s, 0)),
        ],
        out_specs=pl.BlockSpec((B, _BLK_S, D), lambda s: (0, s, 0)),
        out_shape=jax.ShapeDtypeStruct((B, S, D), x.dtype),
    )(x, pos)


_NW = 32
_ROWS_PER_W = _S // _NW
_CHUNK_ROWS = 32
_CHUNK = _CHUNK_ROWS * _D
_N_CHUNKS = _ROWS_PER_W // _CHUNK_ROWS
_UNROLL = 16
_VREGS = _CHUNK // 16


def _sc_body(x_hbm, pos_hbm, out_hbm, x_v, p_v):
    wid = lax.axis_index("s") * 2 + lax.axis_index("c")
    base = wid * (_ROWS_PER_W * _D)

    def chunk_body(ci, carry):
        pos_off = base + ci * _CHUNK
        pltpu.sync_copy(pos_hbm.at[pl.ds(pos_off, _CHUNK)], p_v)
        for b in range(_B):
            x_off = b * (_S * _D) + pos_off
            pltpu.sync_copy(x_hbm.at[pl.ds(x_off, _CHUNK)], x_v)

            def vbody(vi, c2):
                o = vi * (16 * _UNROLL)
                for u in range(_UNROLL):
                    sl = pl.ds(o + u * 16, 16)
                    x_v[sl] = x_v[sl] + p_v[sl]
                return c2

            lax.fori_loop(0, _VREGS // _UNROLL, vbody, 0, unroll=False)
            pltpu.sync_copy(x_v, out_hbm.at[pl.ds(x_off, _CHUNK)])
        return carry

    lax.fori_loop(0, _N_CHUNKS, chunk_body, 0, unroll=False)


@functools.partial(
    pl.kernel,
    mesh=plsc.VectorSubcoreMesh(core_axis_name="c", subcore_axis_name="s"),
    out_type=jax.ShapeDtypeStruct((_B * _S * _D,), jnp.float32),
    scratch_types=[
        pltpu.VMEM((_CHUNK,), jnp.float32),
        pltpu.VMEM((_CHUNK,), jnp.float32),
    ],
)
def _sc_add(x_hbm, pos_hbm, out_hbm, x_v, p_v):
    _sc_body(x_hbm, pos_hbm, out_hbm, x_v, p_v)


def _sc_kernel(x, pos):
    B, S, D = x.shape
    out = _sc_add(x.reshape(-1), pos.reshape(-1))
    return out.reshape(B, S, D)


def kernel(x, pos_table):
    pos = pos_table[: x.shape[1]]
    return _sc_kernel(x, pos)


# speedup vs baseline: 1.8107x; 1.8107x over previous
import functools

import jax
import jax.numpy as jnp
from jax import lax
from jax.experimental import pallas as pl
from jax.experimental.pallas import tpu as pltpu
from jax.experimental.pallas import tpu_sc as plsc

_B, _S, _D = 4, 8192, 1024
_S_SC = 1024
_S_TC = _S - _S_SC


_BLK_S = 512


def _add_block(x_ref, pos_ref, o_ref):
    o_ref[...] = x_ref[...] + pos_ref[...][None, :, :]


def _tc_partial(x, pos):
    B, S, D = x.shape
    grid = (_S_TC // _BLK_S,)
    return pl.pallas_call(
        _add_block,
        grid=grid,
        in_specs=[
            pl.BlockSpec((B, _BLK_S, D), lambda s: (0, s, 0)),
            pl.BlockSpec((_BLK_S, D), lambda s: (s, 0)),
        ],
        out_specs=pl.BlockSpec((B, _BLK_S, D), lambda s: (0, s, 0)),
        out_shape=jax.ShapeDtypeStruct((B, S, D), x.dtype),
    )(x, pos)


_NW = 32
_ROWS_PER_W = _S_SC // _NW
_CHUNK_ROWS = 32
_CHUNK = _CHUNK_ROWS * _D
_N_CHUNKS = _ROWS_PER_W // _CHUNK_ROWS
_UNROLL = 16
_VREGS = _CHUNK // 16


def _sc_body(x_hbm, pos_hbm, out_hbm, x_v, p_v):
    wid = lax.axis_index("s") * 2 + lax.axis_index("c")

    def chunk_body(ci, carry):
        row0 = wid * _ROWS_PER_W + ci * _CHUNK_ROWS
        pos_off = (_S_TC + row0) * _D
        pltpu.sync_copy(pos_hbm.at[pl.ds(pos_off, _CHUNK)], p_v)
        for b in range(_B):
            x_off = b * (_S * _D) + pos_off
            pltpu.sync_copy(x_hbm.at[pl.ds(x_off, _CHUNK)], x_v)

            def vbody(vi, c2):
                o = vi * (16 * _UNROLL)
                for u in range(_UNROLL):
                    sl = pl.ds(o + u * 16, 16)
                    plsc.addupdate(x_v.at[sl], p_v[sl])
                return c2

            lax.fori_loop(0, _VREGS // _UNROLL, vbody, 0, unroll=False)
            out_off = b * (_S_SC * _D) + row0 * _D
            pltpu.sync_copy(x_v, out_hbm.at[pl.ds(out_off, _CHUNK)])
        return carry

    lax.fori_loop(0, _N_CHUNKS, chunk_body, 0, unroll=False)


@functools.partial(
    pl.kernel,
    mesh=plsc.VectorSubcoreMesh(core_axis_name="c", subcore_axis_name="s"),
    out_type=jax.ShapeDtypeStruct((_B * _S_SC * _D,), jnp.float32),
    scratch_types=[
        pltpu.VMEM((_CHUNK,), jnp.float32),
        pltpu.VMEM((_CHUNK,), jnp.float32),
    ],
)
def _sc_partial(x_hbm, pos_hbm, out_hbm, x_v, p_v):
    _sc_body(x_hbm, pos_hbm, out_hbm, x_v, p_v)


def kernel(x, pos_table):
    B, S, D = x.shape
    pos = pos_table[:S]
    out_full = _tc_partial(x, pos)
    out_sc = _sc_partial(x.reshape(-1), pos.reshape(-1))
    return lax.dynamic_update_slice(
        out_full, out_sc.reshape(B, _S_SC, D), (0, _S_TC, 0)
    )
